# Initial kernel scaffold; baseline (speedup 1.0000x reference)
import jax
import jax.numpy as jnp
from jax.experimental import pallas as pl


def kernel(embeddings, labels):
    raise NotImplementedError("write your pallas kernel here")


# speedup vs baseline: 7.3257x; 7.3257x over previous
import jax
import jax.numpy as jnp
from jax.experimental import pallas as pl
from jax.experimental.pallas import tpu as pltpu

_FEATURE_DIM = 64
_NUM_CLASSES = 11
_C_PAD = 16
_ALPHA = 0.9


def _seg_reduce_body(lab_ref, emb_ref, out_ref, acc_ref, cnt_ref):
    i = pl.program_id(0)
    nc = pl.num_programs(0)

    @pl.when(i == 0)
    def _init():
        acc_ref[...] = jnp.zeros_like(acc_ref)
        cnt_ref[...] = jnp.zeros_like(cnt_ref)

    classes = jax.lax.broadcasted_iota(jnp.int32, (_C_PAD, 1), 0)
    B = emb_ref.shape[0]
    for b in range(B):
        lab = lab_ref[0, b, :]
        onehot = (lab[None, :] == classes).astype(jnp.float32)
        emb = emb_ref[b]
        part = jax.lax.dot_general(
            onehot, emb, (((1,), (1,)), ((), ())),
            preferred_element_type=jnp.float32)
        acc_ref[b] += part
        cnt_ref[b] += jnp.sum(onehot, axis=1, keepdims=True)

    @pl.when(i == nc - 1)
    def _finalize():
        s0, s1 = acc_ref[0], acc_ref[1]
        c0, c1 = cnt_ref[0], cnt_ref[1]
        m0 = s0 / jnp.maximum(c0, 1.0)
        m1 = s1 / jnp.maximum(c1, 1.0)
        p0 = c0 > 0.0
        p1 = c1 > 0.0
        upd = jnp.where(p0, _ALPHA * m0 + (1.0 - _ALPHA) * m1, m1)
        out_ref[...] = jnp.where(p1, upd, jnp.where(p0, m0, 0.0))


def kernel(embeddings, labels):
    B, F, D, H, W = embeddings.shape
    V = D * H * W
    emb3 = embeddings.reshape(B, F, V)
    lab = labels.reshape(B, V).astype(jnp.int32)

    Vc = 16384
    assert V % Vc == 0
    nc = V // Vc
    lab_r = lab.reshape(B, nc, Vc).transpose(1, 0, 2)

    out = pl.pallas_call(
        _seg_reduce_body,
        grid=(nc,),
        in_specs=[
            pl.BlockSpec((1, B, Vc), lambda i: (i, 0, 0)),
            pl.BlockSpec((B, F, Vc), lambda i: (0, 0, i)),
        ],
        out_specs=pl.BlockSpec((_C_PAD, F), lambda i: (0, 0)),
        out_shape=jax.ShapeDtypeStruct((_C_PAD, F), jnp.float32),
        scratch_shapes=[
            pltpu.VMEM((B, _C_PAD, F), jnp.float32),
            pltpu.VMEM((B, _C_PAD, 1), jnp.float32),
        ],
    )(lab_r, emb3)
    return out[:_NUM_CLASSES]


# speedup vs baseline: 7.3458x; 1.0027x over previous
import jax
import jax.numpy as jnp
from jax.experimental import pallas as pl
from jax.experimental.pallas import tpu as pltpu

_FEATURE_DIM = 64
_NUM_CLASSES = 11
_C_PAD = 16
_ALPHA = 0.9


def _seg_reduce_body(lab_ref, emb_ref, out_ref, acc_ref, cnt_ref):
    i = pl.program_id(0)
    nc = pl.num_programs(0)

    @pl.when(i == 0)
    def _init():
        acc_ref[...] = jnp.zeros_like(acc_ref)
        cnt_ref[...] = jnp.zeros_like(cnt_ref)

    classes = jax.lax.broadcasted_iota(jnp.int32, (_C_PAD, 1), 0)
    B = emb_ref.shape[0]
    for b in range(B):
        lab = lab_ref[0, b, :]
        onehot = (lab[None, :] == classes).astype(jnp.float32)
        emb = emb_ref[b]
        part = jax.lax.dot_general(
            onehot, emb, (((1,), (1,)), ((), ())),
            preferred_element_type=jnp.float32)
        acc_ref[b] += part
        cnt_ref[b] += jnp.sum(onehot, axis=1, keepdims=True)

    @pl.when(i == nc - 1)
    def _finalize():
        s0, s1 = acc_ref[0], acc_ref[1]
        c0, c1 = cnt_ref[0], cnt_ref[1]
        m0 = s0 / jnp.maximum(c0, 1.0)
        m1 = s1 / jnp.maximum(c1, 1.0)
        p0 = c0 > 0.0
        p1 = c1 > 0.0
        upd = jnp.where(p0, _ALPHA * m0 + (1.0 - _ALPHA) * m1, m1)
        out_ref[...] = jnp.where(p1, upd, jnp.where(p0, m0, 0.0))


def kernel(embeddings, labels):
    B, F, D, H, W = embeddings.shape
    V = D * H * W
    emb3 = embeddings.reshape(B, F, V)
    lab = labels.reshape(B, V).astype(jnp.int32)

    Vc = 27648
    assert V % Vc == 0
    nc = V // Vc
    lab_r = lab.reshape(B, nc, Vc).transpose(1, 0, 2)

    out = pl.pallas_call(
        _seg_reduce_body,
        grid=(nc,),
        in_specs=[
            pl.BlockSpec((1, B, Vc), lambda i: (i, 0, 0)),
            pl.BlockSpec((B, F, Vc), lambda i: (0, 0, i)),
        ],
        out_specs=pl.BlockSpec((_C_PAD, F), lambda i: (0, 0)),
        out_shape=jax.ShapeDtypeStruct((_C_PAD, F), jnp.float32),
        scratch_shapes=[
            pltpu.VMEM((B, _C_PAD, F), jnp.float32),
            pltpu.VMEM((B, _C_PAD, 1), jnp.float32),
        ],
    )(lab_r, emb3)
    return out[:_NUM_CLASSES]
